# baseline (speedup 1.0000x reference)
import functools

import jax
import jax.numpy as jnp
from jax import lax
from jax.experimental import pallas as pl
from jax.experimental.pallas import tpu as pltpu
from jax.experimental.pallas import tpu_sc as plsc

N = 10000
E = 320000
L = 50000
D_IN = 128
H1 = 64
H2 = 32

NC = 2
NS = 16
NW = NC * NS
NP = 10240
RPT = NP // NS

WIN = 64
CH = 8
NWIN = 160
EPW = NWIN * WIN
EPAD = EPW * NW

LPW = 1600
LPAD = LPW * NW

RB = 2048
GRID = -(-N // RB)


def _tca_body(x_ref, w1_ref, wl_ref, bl_ref, a_ref, h_ref, xl_ref, asd_ref):
    xx = x_ref[...]
    h = jnp.dot(xx, w1_ref[...], preferred_element_type=jnp.float32)
    h_ref[...] = h
    xl_ref[...] = jnp.dot(xx, wl_ref[...], preferred_element_type=jnp.float32) + bl_ref[...]
    asd_ref[...] = jnp.dot(h, a_ref[...], preferred_element_type=jnp.float32)


def _tca(x, W1, Wl1, bl1, A1):
    return pl.pallas_call(
        _tca_body,
        grid=(GRID,),
        in_specs=[
            pl.BlockSpec((RB, D_IN), lambda i: (i, 0)),
            pl.BlockSpec((D_IN, H1), lambda i: (0, 0)),
            pl.BlockSpec((D_IN, H1), lambda i: (0, 0)),
            pl.BlockSpec((1, H1), lambda i: (0, 0)),
            pl.BlockSpec((H1, 8), lambda i: (0, 0)),
        ],
        out_specs=[
            pl.BlockSpec((RB, H1), lambda i: (i, 0)),
            pl.BlockSpec((RB, H1), lambda i: (i, 0)),
            pl.BlockSpec((RB, 8), lambda i: (i, 0)),
        ],
        out_shape=[
            jax.ShapeDtypeStruct((N, H1), jnp.float32),
            jax.ShapeDtypeStruct((N, H1), jnp.float32),
            jax.ShapeDtypeStruct((N, 8), jnp.float32),
        ],
    )(x, W1, Wl1, bl1, A1)


def _make_edge_kernel(H):
    HR = H + 16
    QH = H // 16
    RT = N // NS
    mesh = plsc.VectorSubcoreMesh(core_axis_name="c", subcore_axis_name="s")

    @functools.partial(
        pl.kernel,
        mesh=mesh,
        out_type=jax.ShapeDtypeStruct((NC, N, HR), jnp.float32),
        scratch_types=[
            pltpu.VMEM((N,), jnp.float32),
            pltpu.VMEM((N,), jnp.float32),
            pltpu.VMEM((2, CH, WIN), jnp.int32),
            pltpu.VMEM((2, CH, WIN), jnp.int32),
            pltpu.VMEM((WIN,), jnp.float32),
            pltpu.VMEM((2, WIN, H), jnp.float32),
            pltpu.VMEM((2, WIN, HR), jnp.float32),
            pltpu.VMEM_SHARED((N, H), jnp.float32),
            pltpu.VMEM_SHARED((N, HR), jnp.float32),
            pltpu.SemaphoreType.DMA,
            pltpu.SemaphoreType.DMA,
            pltpu.SemaphoreType.DMA,
            pltpu.SemaphoreType.DMA,
        ],
        compiler_params=pltpu.CompilerParams(
            needs_layout_passes=False, use_tc_tiling_on_sc=False),
    )
    def edge_kernel(h_hbm, as_hbm, ad_hbm, src_hbm, dst_hbm, num_hbm,
                    as_v, ad_v, src_v, dst_v, ex_v, rows_g, rows_v,
                    h_s, num_s, gsem0, gsem1, ssem0, ssem1):
        gsem = (gsem0, gsem1)
        ssem = (ssem0, ssem1)
        core = lax.axis_index("c")
        sub = lax.axis_index("s")
        wid = sub * NC + core
        base_win = wid * NWIN

        pltpu.sync_copy(as_hbm, as_v)
        pltpu.sync_copy(ad_hbm, ad_v)
        pltpu.sync_copy(h_hbm.at[pl.ds(sub * RT, RT)],
                        h_s.at[pl.ds(sub * RT, RT)])

        zv = jnp.zeros((16,), jnp.float32)

        def zrow_body(r, _):
            for q in range(HR // 16):
                rows_v[0, r, pl.ds(q * 16, 16)] = zv
            return 0

        lax.fori_loop(0, WIN, zrow_body, 0)
        for k in range(RT // 25):
            pltpu.sync_copy(rows_v.at[0, pl.ds(0, 25)],
                            num_s.at[pl.ds(sub * RT + k * 25, 25)])
        plsc.subcore_barrier()

        lane0 = lax.iota(jnp.int32, 16) == 0
        ebase = wid * EPW

        def _issue_gather(w, s):
            cs = lax.rem(lax.div(w, CH), 2)
            wm = lax.rem(w, CH)
            pltpu.async_copy(h_s.at[src_v.at[cs, wm]], rows_g.at[s], gsem[s])

        def _wait_gather(s):
            pltpu.make_async_copy(
                h_hbm.at[pl.ds(0, WIN)], rows_g.at[s], gsem[s]).wait()

        def _drain_scatter(s):
            pltpu.make_async_copy(
                num_hbm.at[0, pl.ds(0, WIN)], rows_v.at[s], ssem[s]).wait()

        def _load_chunk(c):
            cs = lax.rem(c, 2)
            pltpu.sync_copy(
                src_hbm.at[pl.ds(base_win + c * CH, CH)], src_v.at[cs])
            pltpu.sync_copy(
                dst_hbm.at[pl.ds(base_win + c * CH, CH)], dst_v.at[cs])

        _load_chunk(0)
        _issue_gather(0, 0)

        def pair_body(t, _):
            for s in range(2):
                w = 2 * t + s
                _wait_gather(s)
                cs_w = lax.rem(lax.div(w, CH), 2)
                wm_w = lax.rem(w, CH)

                def grp_body(g, _):
                    off = pl.multiple_of(g * 16, 16)
                    rid = off + lax.iota(jnp.int32, 16)
                    s16 = src_v[cs_w, wm_w, pl.ds(off, 16)]
                    d16 = dst_v[cs_w, wm_w, pl.ds(off, 16)]
                    a_s = plsc.load_gather(as_v, [s16])
                    a_d = plsc.load_gather(ad_v, [d16])
                    e = a_s + a_d
                    e = jnp.where(e >= 0.0, e, e * jnp.float32(0.2))
                    ex = jnp.exp(e)
                    gid = ebase + w * WIN + rid
                    ex = jnp.where(gid < E, ex, jnp.float32(0.0))
                    ex_v[pl.ds(off, 16)] = ex
                    for j in range(16):
                        row = g * 16 + j
                        eb = plsc.load_gather(
                            ex_v, [jnp.full((16,), row, jnp.int32)])
                        for q in range(QH):
                            rows_v[s, row, pl.ds(q * 16, 16)] = (
                                rows_g[s, row, pl.ds(q * 16, 16)] * eb)
                        rows_v[s, row, pl.ds(H, 16)] = jnp.where(
                            lane0, eb, 0.0)
                    return 0

                lax.fori_loop(0, WIN // 16, grp_body, 0)
                pltpu.async_copy(rows_v.at[s], num_s.at[dst_v.at[cs_w, wm_w]],
                                 ssem[s], add=True)

                @pl.when(w + 1 < NWIN)
                def _prefetch():
                    @pl.when(lax.rem(w + 1, CH) == 0)
                    def _chunk():
                        _load_chunk(lax.div(w + 1, CH))

                    @pl.when(w >= 1)
                    def _drain():
                        _drain_scatter(1 - s)

                    _issue_gather(w + 1, 1 - s)
            return 0

        lax.fori_loop(0, NWIN // 2, pair_body, 0)
        _drain_scatter(0)
        _drain_scatter(1)
        plsc.subcore_barrier()
        pltpu.sync_copy(num_s.at[pl.ds(sub * RT, RT)],
                        num_hbm.at[core, pl.ds(sub * RT, RT)])

    return edge_kernel


_edge1 = _make_edge_kernel(H1)
_edge2 = _make_edge_kernel(H2)


def _tcb_body(np_ref, xl_ref, b1_ref, d_ref, w2_ref, wl2_ref,
              bl2_ref, a2_ref, cur_ref, h2_ref, xl2_ref, asd2_ref):
    num = np_ref[0] + np_ref[1]
    h = num[:, 0:H1] / (num[:, H1:H1 + 1] + 1e-16) + b1_ref[...]
    d = d_ref[...]
    cur1 = d * h + (1.0 - d) * xl_ref[...]
    cur_ref[...] = cur1
    h2 = jnp.dot(cur1, w2_ref[...], preferred_element_type=jnp.float32)
    h2_ref[...] = h2
    xl2_ref[...] = jnp.dot(cur1, wl2_ref[...], preferred_element_type=jnp.float32) + bl2_ref[...]
    asd2_ref[...] = jnp.dot(h2, a2_ref[...], preferred_element_type=jnp.float32)


def _tcb(num1, xlin1, b1, delta1, W2, Wl2, bl2, A2):
    return pl.pallas_call(
        _tcb_body,
        grid=(GRID,),
        in_specs=[
            pl.BlockSpec((NC, RB, H1 + 16), lambda i: (0, i, 0)),
            pl.BlockSpec((RB, H1), lambda i: (i, 0)),
            pl.BlockSpec((1, H1), lambda i: (0, 0)),
            pl.BlockSpec((1, 1), lambda i: (0, 0)),
            pl.BlockSpec((H1, H2), lambda i: (0, 0)),
            pl.BlockSpec((H1, H2), lambda i: (0, 0)),
            pl.BlockSpec((1, H2), lambda i: (0, 0)),
            pl.BlockSpec((H2, 8), lambda i: (0, 0)),
        ],
        out_specs=[
            pl.BlockSpec((RB, H1), lambda i: (i, 0)),
            pl.BlockSpec((RB, H2), lambda i: (i, 0)),
            pl.BlockSpec((RB, H2), lambda i: (i, 0)),
            pl.BlockSpec((RB, 8), lambda i: (i, 0)),
        ],
        out_shape=[
            jax.ShapeDtypeStruct((N, H1), jnp.float32),
            jax.ShapeDtypeStruct((N, H2), jnp.float32),
            jax.ShapeDtypeStruct((N, H2), jnp.float32),
            jax.ShapeDtypeStruct((N, 8), jnp.float32),
        ],
    )(num1, xlin1, b1, delta1, W2, Wl2, bl2, A2)


def _tcc_body(np_ref, xl_ref, b2_ref, d_ref, wp_ref, bp_ref,
              cur_ref, pq_ref):
    num = np_ref[0] + np_ref[1]
    h = num[:, 0:H2] / (num[:, H2:H2 + 1] + 1e-16) + b2_ref[...]
    d = d_ref[...]
    cur2 = d * h + (1.0 - d) * xl_ref[...]
    cur_ref[...] = cur2
    t1 = jnp.dot(cur2, wp_ref[0:H2, :], preferred_element_type=jnp.float32)
    t2 = jnp.dot(cur2, wp_ref[H2:2 * H2, :], preferred_element_type=jnp.float32)
    bp = bp_ref[...]
    c = bp[:, 0:1] + bp[:, 1:2]
    p = t1[:, 0:1] + t1[:, 1:2] + c
    q = t2[:, 0:1] + t2[:, 1:2]
    pq_ref[...] = jnp.concatenate(
        [p, q, jnp.zeros((p.shape[0], 6), jnp.float32)], axis=1)


def _tcc(num2, xlin2, b2, delta2, Wp, bp):
    return pl.pallas_call(
        _tcc_body,
        grid=(GRID,),
        in_specs=[
            pl.BlockSpec((NC, RB, H2 + 16), lambda i: (0, i, 0)),
            pl.BlockSpec((RB, H2), lambda i: (i, 0)),
            pl.BlockSpec((1, H2), lambda i: (0, 0)),
            pl.BlockSpec((1, 1), lambda i: (0, 0)),
            pl.BlockSpec((2 * H2, 2), lambda i: (0, 0)),
            pl.BlockSpec((1, 2), lambda i: (0, 0)),
        ],
        out_specs=[
            pl.BlockSpec((RB, H2), lambda i: (i, 0)),
            pl.BlockSpec((RB, 8), lambda i: (i, 0)),
        ],
        out_shape=[
            jax.ShapeDtypeStruct((N, H2), jnp.float32),
            jax.ShapeDtypeStruct((N, 8), jnp.float32),
        ],
    )(num2, xlin2, b2, delta2, Wp, bp)


def _make_score_kernel():
    mesh = plsc.VectorSubcoreMesh(core_axis_name="c", subcore_axis_name="s")

    @functools.partial(
        pl.kernel,
        mesh=mesh,
        out_type=jax.ShapeDtypeStruct((NW, 1, LPW), jnp.float32),
        scratch_types=[
            pltpu.VMEM((N,), jnp.float32),
            pltpu.VMEM((N,), jnp.float32),
            pltpu.VMEM((1, LPW), jnp.int32),
            pltpu.VMEM((1, LPW), jnp.int32),
            pltpu.VMEM((1, LPW), jnp.float32),
        ],
        compiler_params=pltpu.CompilerParams(
            needs_layout_passes=False, use_tc_tiling_on_sc=False),
    )
    def score_kernel(p_hbm, q_hbm, i0_hbm, i1_hbm, out_hbm,
                     p_v, q_v, i0_v, i1_v, out_v):
        core = lax.axis_index("c")
        sub = lax.axis_index("s")
        wid = sub * NC + core
        pltpu.sync_copy(p_hbm, p_v)
        pltpu.sync_copy(q_hbm, q_v)
        pltpu.sync_copy(i0_hbm.at[wid], i0_v)
        pltpu.sync_copy(i1_hbm.at[wid], i1_v)

        def body(g, _):
            off = pl.multiple_of(g * 16, 16)
            idx0 = i0_v[0, pl.ds(off, 16)]
            idx1 = i1_v[0, pl.ds(off, 16)]
            out_v[0, pl.ds(off, 16)] = (
                plsc.load_gather(p_v, [idx0]) + plsc.load_gather(q_v, [idx1]))
            return 0

        lax.fori_loop(0, LPW // 16, body, 0)
        pltpu.sync_copy(out_v, out_hbm.at[wid])

    return score_kernel


_score = _make_score_kernel()


def kernel(x, edge_index, edge_label_index, snap, W_lin1, b_lin1, W1, a_src1,
           a_dst1, b1, Wsem1, bsem1, qsem1, delta1, W_lin2, b_lin2, W2,
           a_src2, a_dst2, b2, Wsem2, bsem2, qsem2, delta2, W_post, b_post):
    f32 = jnp.float32
    pad_e = (jnp.arange(EPAD - E, dtype=jnp.int32) * 37) % N
    src2d = jnp.concatenate([edge_index[0], pad_e]).reshape(NW * NWIN, WIN)
    dst2d = jnp.concatenate([edge_index[1], pad_e]).reshape(NW * NWIN, WIN)
    pad_l = (jnp.arange(LPAD - L, dtype=jnp.int32) * 53) % N
    eli0 = jnp.concatenate([edge_label_index[0], pad_l]).reshape(NW, 1, LPW)
    eli1 = jnp.concatenate([edge_label_index[1], pad_l]).reshape(NW, 1, LPW)

    A1 = jnp.zeros((H1, 8), f32).at[:, 0].set(a_src1).at[:, 1].set(a_dst1)
    A2 = jnp.zeros((H2, 8), f32).at[:, 0].set(a_src2).at[:, 1].set(a_dst2)

    h1, xlin1, asd1 = _tca(x, W1, W_lin1, b_lin1.reshape(1, H1), A1)
    num1 = _edge1(h1, asd1[:, 0], asd1[:, 1], src2d, dst2d)
    cur1, h2, xlin2, asd2 = _tcb(num1, xlin1, b1.reshape(1, H1),
                                 delta1.reshape(1, 1), W2, W_lin2,
                                 b_lin2.reshape(1, H2), A2)
    num2 = _edge2(h2, asd2[:, 0], asd2[:, 1], src2d, dst2d)
    cur2, pq = _tcc(num2, xlin2, b2.reshape(1, H2),
                    delta2.reshape(1, 1), W_post, b_post.reshape(1, 2))
    scores2d = _score(pq[:, 0], pq[:, 1], eli0, eli1)
    scores = scores2d.reshape(-1)[:L]
    return (scores, cur1, cur2)


# speedup vs baseline: 1.1361x; 1.1361x over previous
import functools

import jax
import jax.numpy as jnp
from jax import lax
from jax.experimental import pallas as pl
from jax.experimental.pallas import tpu as pltpu
from jax.experimental.pallas import tpu_sc as plsc

N = 10000
E = 320000
L = 50000
D_IN = 128
H1 = 64
H2 = 32

NC = 2
NS = 16
NW = NC * NS
NP = 10240
RPT = NP // NS

WIN = 128
CH = 8
NWIN = 80
EPW = NWIN * WIN
EPAD = EPW * NW

LPW = 1600
LPAD = LPW * NW

RB = 2048
GRID = -(-N // RB)


def _tca_body(x_ref, w1_ref, wl_ref, bl_ref, ax_ref, ad_ref,
              hh_ref, xl_ref, adv_ref):
    xx = x_ref[...]
    h = jnp.dot(xx, w1_ref[...], preferred_element_type=jnp.float32)
    asd = jnp.dot(h, ax_ref[...], preferred_element_type=jnp.float32)
    hh_ref[...] = jnp.concatenate([h, asd], axis=1)
    xl_ref[...] = jnp.dot(xx, wl_ref[...], preferred_element_type=jnp.float32) + bl_ref[...]
    adv_ref[...] = jnp.dot(h, ad_ref[...], preferred_element_type=jnp.float32)


def _tca(x, W1, Wl1, bl1, A1x, Ad1):
    return pl.pallas_call(
        _tca_body,
        grid=(GRID,),
        in_specs=[
            pl.BlockSpec((RB, D_IN), lambda i: (i, 0)),
            pl.BlockSpec((D_IN, H1), lambda i: (0, 0)),
            pl.BlockSpec((D_IN, H1), lambda i: (0, 0)),
            pl.BlockSpec((1, H1), lambda i: (0, 0)),
            pl.BlockSpec((H1, 16), lambda i: (0, 0)),
            pl.BlockSpec((H1, 8), lambda i: (0, 0)),
        ],
        out_specs=[
            pl.BlockSpec((RB, H1 + 16), lambda i: (i, 0)),
            pl.BlockSpec((RB, H1), lambda i: (i, 0)),
            pl.BlockSpec((RB, 8), lambda i: (i, 0)),
        ],
        out_shape=[
            jax.ShapeDtypeStruct((N, H1 + 16), jnp.float32),
            jax.ShapeDtypeStruct((N, H1), jnp.float32),
            jax.ShapeDtypeStruct((N, 8), jnp.float32),
        ],
    )(x, W1, Wl1, bl1, A1x, Ad1)


def _make_edge_kernel(H):
    HR = H + 16
    QH = H // 16
    RT = N // NS
    mesh = plsc.VectorSubcoreMesh(core_axis_name="c", subcore_axis_name="s")

    @functools.partial(
        pl.kernel,
        mesh=mesh,
        out_type=jax.ShapeDtypeStruct((NC, N, HR), jnp.float32),
        scratch_types=[
            pltpu.VMEM((2, CH, WIN), jnp.int32),
            pltpu.VMEM((2, CH, WIN), jnp.int32),
            pltpu.VMEM((WIN,), jnp.float32),
            pltpu.VMEM((2, WIN, HR), jnp.float32),
            pltpu.VMEM((2, WIN), jnp.float32),
            pltpu.VMEM_SHARED((N, HR), jnp.float32),
            pltpu.VMEM_SHARED((N,), jnp.float32),
            pltpu.VMEM_SHARED((N, HR), jnp.float32),
            pltpu.SemaphoreType.DMA,
            pltpu.SemaphoreType.DMA,
            pltpu.SemaphoreType.DMA,
            pltpu.SemaphoreType.DMA,
        ],
        compiler_params=pltpu.CompilerParams(
            needs_layout_passes=False, use_tc_tiling_on_sc=False),
    )
    def edge_kernel(hh_hbm, ad_hbm, src_hbm, dst_hbm, num_hbm,
                    src_v, dst_v, ex_v, rows_v, adw_v,
                    h_s, ad_s, num_s, gsem0, gsem1, ssem0, ssem1):
        gsem = (gsem0, gsem1)
        ssem = (ssem0, ssem1)
        core = lax.axis_index("c")
        sub = lax.axis_index("s")
        wid = sub * NC + core
        base_win = wid * NWIN

        pltpu.sync_copy(hh_hbm.at[pl.ds(sub * RT, RT)],
                        h_s.at[pl.ds(sub * RT, RT)])

        @pl.when(sub == 0)
        def _stage_ad():
            pltpu.sync_copy(ad_hbm, ad_s)

        zv = jnp.zeros((16,), jnp.float32)

        def zrow_body(r, _):
            for q in range(HR // 16):
                rows_v[0, r, pl.ds(q * 16, 16)] = zv
            return 0

        lax.fori_loop(0, WIN, zrow_body, 0)
        for k in range(RT // 125):
            pltpu.sync_copy(rows_v.at[0, pl.ds(0, 125)],
                            num_s.at[pl.ds(sub * RT + k * 125, 125)])
        plsc.subcore_barrier()

        lane0 = lax.iota(jnp.int32, 16) == 0
        ebase = wid * EPW

        def _issue_gather(w, s):
            cs = lax.rem(lax.div(w, CH), 2)
            wm = lax.rem(w, CH)
            pltpu.async_copy(h_s.at[src_v.at[cs, wm]], rows_v.at[s], gsem[s])
            pltpu.async_copy(ad_s.at[dst_v.at[cs, wm]], adw_v.at[s], gsem[s])

        def _wait_gather(s):
            pltpu.make_async_copy(
                hh_hbm.at[pl.ds(0, WIN)], rows_v.at[s], gsem[s]).wait()
            pltpu.make_async_copy(
                ad_hbm.at[pl.ds(0, WIN)], adw_v.at[s], gsem[s]).wait()

        def _drain_scatter(s):
            pltpu.make_async_copy(
                num_hbm.at[0, pl.ds(0, WIN)], rows_v.at[s], ssem[s]).wait()

        def _load_chunk(c):
            cs = lax.rem(c, 2)
            pltpu.sync_copy(
                src_hbm.at[pl.ds(base_win + c * CH, CH)], src_v.at[cs])
            pltpu.sync_copy(
                dst_hbm.at[pl.ds(base_win + c * CH, CH)], dst_v.at[cs])

        _load_chunk(0)
        _issue_gather(0, 0)

        def pair_body(t, _):
            for s in range(2):
                w = 2 * t + s
                _wait_gather(s)
                cs_w = lax.rem(lax.div(w, CH), 2)
                wm_w = lax.rem(w, CH)

                def grp_body(g, _):
                    off = pl.multiple_of(g * 16, 16)
                    rid = off + lax.iota(jnp.int32, 16)
                    a_s = plsc.load_gather(
                        rows_v, [jnp.full((16,), s, jnp.int32), rid,
                                 jnp.full((16,), H, jnp.int32)])
                    a_d = adw_v[s, pl.ds(off, 16)]
                    e = a_s + a_d
                    e = jnp.where(e >= 0.0, e, e * jnp.float32(0.2))
                    ex = jnp.exp(e)
                    gid = ebase + w * WIN + rid
                    ex = jnp.where(gid < E, ex, jnp.float32(0.0))
                    ex_v[pl.ds(off, 16)] = ex
                    for j in range(16):
                        row = g * 16 + j
                        eb = plsc.load_gather(
                            ex_v, [jnp.full((16,), row, jnp.int32)])
                        for q in range(QH):
                            rows_v[s, row, pl.ds(q * 16, 16)] = (
                                rows_v[s, row, pl.ds(q * 16, 16)] * eb)
                        rows_v[s, row, pl.ds(H, 16)] = jnp.where(
                            lane0, eb, 0.0)
                    return 0

                lax.fori_loop(0, WIN // 16, grp_body, 0)
                pltpu.async_copy(rows_v.at[s], num_s.at[dst_v.at[cs_w, wm_w]],
                                 ssem[s], add=True)

                @pl.when(w + 1 < NWIN)
                def _prefetch():
                    @pl.when(lax.rem(w + 1, CH) == 0)
                    def _chunk():
                        _load_chunk(lax.div(w + 1, CH))

                    @pl.when(w >= 1)
                    def _drain():
                        _drain_scatter(1 - s)

                    _issue_gather(w + 1, 1 - s)
            return 0

        lax.fori_loop(0, NWIN // 2, pair_body, 0)
        _drain_scatter(0)
        _drain_scatter(1)
        plsc.subcore_barrier()
        pltpu.sync_copy(num_s.at[pl.ds(sub * RT, RT)],
                        num_hbm.at[core, pl.ds(sub * RT, RT)])

    return edge_kernel


_edge1 = _make_edge_kernel(H1)
_edge2 = _make_edge_kernel(H2)


def _tcb_body(np_ref, xl_ref, b1_ref, d_ref, w2_ref, wl2_ref,
              bl2_ref, a2x_ref, ad2_ref, cur_ref, hh2_ref, xl2_ref, adv2_ref):
    num = np_ref[0] + np_ref[1]
    h = num[:, 0:H1] / (num[:, H1:H1 + 1] + 1e-16) + b1_ref[...]
    d = d_ref[...]
    cur1 = d * h + (1.0 - d) * xl_ref[...]
    cur_ref[...] = cur1
    h2 = jnp.dot(cur1, w2_ref[...], preferred_element_type=jnp.float32)
    asd2 = jnp.dot(h2, a2x_ref[...], preferred_element_type=jnp.float32)
    hh2_ref[...] = jnp.concatenate([h2, asd2], axis=1)
    xl2_ref[...] = jnp.dot(cur1, wl2_ref[...], preferred_element_type=jnp.float32) + bl2_ref[...]
    adv2_ref[...] = jnp.dot(h2, ad2_ref[...], preferred_element_type=jnp.float32)


def _tcb(num1, xlin1, b1, delta1, W2, Wl2, bl2, A2x, Ad2):
    return pl.pallas_call(
        _tcb_body,
        grid=(GRID,),
        in_specs=[
            pl.BlockSpec((NC, RB, H1 + 16), lambda i: (0, i, 0)),
            pl.BlockSpec((RB, H1), lambda i: (i, 0)),
            pl.BlockSpec((1, H1), lambda i: (0, 0)),
            pl.BlockSpec((1, 1), lambda i: (0, 0)),
            pl.BlockSpec((H1, H2), lambda i: (0, 0)),
            pl.BlockSpec((H1, H2), lambda i: (0, 0)),
            pl.BlockSpec((1, H2), lambda i: (0, 0)),
            pl.BlockSpec((H2, 16), lambda i: (0, 0)),
            pl.BlockSpec((H2, 8), lambda i: (0, 0)),
        ],
        out_specs=[
            pl.BlockSpec((RB, H1), lambda i: (i, 0)),
            pl.BlockSpec((RB, H2 + 16), lambda i: (i, 0)),
            pl.BlockSpec((RB, H2), lambda i: (i, 0)),
            pl.BlockSpec((RB, 8), lambda i: (i, 0)),
        ],
        out_shape=[
            jax.ShapeDtypeStruct((N, H1), jnp.float32),
            jax.ShapeDtypeStruct((N, H2 + 16), jnp.float32),
            jax.ShapeDtypeStruct((N, H2), jnp.float32),
            jax.ShapeDtypeStruct((N, 8), jnp.float32),
        ],
    )(num1, xlin1, b1, delta1, W2, Wl2, bl2, A2x, Ad2)


def _tcc_body(np_ref, xl_ref, b2_ref, d_ref, wp_ref, bp_ref,
              cur_ref, pq_ref):
    num = np_ref[0] + np_ref[1]
    h = num[:, 0:H2] / (num[:, H2:H2 + 1] + 1e-16) + b2_ref[...]
    d = d_ref[...]
    cur2 = d * h + (1.0 - d) * xl_ref[...]
    cur_ref[...] = cur2
    t1 = jnp.dot(cur2, wp_ref[0:H2, :], preferred_element_type=jnp.float32)
    t2 = jnp.dot(cur2, wp_ref[H2:2 * H2, :], preferred_element_type=jnp.float32)
    bp = bp_ref[...]
    c = bp[:, 0:1] + bp[:, 1:2]
    p = t1[:, 0:1] + t1[:, 1:2] + c
    q = t2[:, 0:1] + t2[:, 1:2]
    pq_ref[...] = jnp.concatenate(
        [p, q, jnp.zeros((p.shape[0], 6), jnp.float32)], axis=1)


def _tcc(num2, xlin2, b2, delta2, Wp, bp):
    return pl.pallas_call(
        _tcc_body,
        grid=(GRID,),
        in_specs=[
            pl.BlockSpec((NC, RB, H2 + 16), lambda i: (0, i, 0)),
            pl.BlockSpec((RB, H2), lambda i: (i, 0)),
            pl.BlockSpec((1, H2), lambda i: (0, 0)),
            pl.BlockSpec((1, 1), lambda i: (0, 0)),
            pl.BlockSpec((2 * H2, 2), lambda i: (0, 0)),
            pl.BlockSpec((1, 2), lambda i: (0, 0)),
        ],
        out_specs=[
            pl.BlockSpec((RB, H2), lambda i: (i, 0)),
            pl.BlockSpec((RB, 8), lambda i: (i, 0)),
        ],
        out_shape=[
            jax.ShapeDtypeStruct((N, H2), jnp.float32),
            jax.ShapeDtypeStruct((N, 8), jnp.float32),
        ],
    )(num2, xlin2, b2, delta2, Wp, bp)


def _make_score_kernel():
    mesh = plsc.VectorSubcoreMesh(core_axis_name="c", subcore_axis_name="s")

    @functools.partial(
        pl.kernel,
        mesh=mesh,
        out_type=jax.ShapeDtypeStruct((NW, 1, LPW), jnp.float32),
        scratch_types=[
            pltpu.VMEM((N,), jnp.float32),
            pltpu.VMEM((N,), jnp.float32),
            pltpu.VMEM((1, LPW), jnp.int32),
            pltpu.VMEM((1, LPW), jnp.int32),
            pltpu.VMEM((1, LPW), jnp.float32),
        ],
        compiler_params=pltpu.CompilerParams(
            needs_layout_passes=False, use_tc_tiling_on_sc=False),
    )
    def score_kernel(p_hbm, q_hbm, i0_hbm, i1_hbm, out_hbm,
                     p_v, q_v, i0_v, i1_v, out_v):
        core = lax.axis_index("c")
        sub = lax.axis_index("s")
        wid = sub * NC + core
        pltpu.sync_copy(p_hbm, p_v)
        pltpu.sync_copy(q_hbm, q_v)
        pltpu.sync_copy(i0_hbm.at[wid], i0_v)
        pltpu.sync_copy(i1_hbm.at[wid], i1_v)

        def body(g, _):
            off = pl.multiple_of(g * 16, 16)
            idx0 = i0_v[0, pl.ds(off, 16)]
            idx1 = i1_v[0, pl.ds(off, 16)]
            out_v[0, pl.ds(off, 16)] = (
                plsc.load_gather(p_v, [idx0]) + plsc.load_gather(q_v, [idx1]))
            return 0

        lax.fori_loop(0, LPW // 16, body, 0)
        pltpu.sync_copy(out_v, out_hbm.at[wid])

    return score_kernel


_score = _make_score_kernel()


def kernel(x, edge_index, edge_label_index, snap, W_lin1, b_lin1, W1, a_src1,
           a_dst1, b1, Wsem1, bsem1, qsem1, delta1, W_lin2, b_lin2, W2,
           a_src2, a_dst2, b2, Wsem2, bsem2, qsem2, delta2, W_post, b_post):
    f32 = jnp.float32
    pad_e = (jnp.arange(EPAD - E, dtype=jnp.int32) * 37) % N
    src2d = jnp.concatenate([edge_index[0], pad_e]).reshape(NW * NWIN, WIN)
    dst2d = jnp.concatenate([edge_index[1], pad_e]).reshape(NW * NWIN, WIN)
    pad_l = (jnp.arange(LPAD - L, dtype=jnp.int32) * 53) % N
    eli0 = jnp.concatenate([edge_label_index[0], pad_l]).reshape(NW, 1, LPW)
    eli1 = jnp.concatenate([edge_label_index[1], pad_l]).reshape(NW, 1, LPW)

    A1x = jnp.zeros((H1, 16), f32).at[:, 0].set(a_src1)
    Ad1 = jnp.zeros((H1, 8), f32).at[:, 0].set(a_dst1)
    A2x = jnp.zeros((H2, 16), f32).at[:, 0].set(a_src2)
    Ad2 = jnp.zeros((H2, 8), f32).at[:, 0].set(a_dst2)

    hh1, xlin1, adv1 = _tca(x, W1, W_lin1, b_lin1.reshape(1, H1), A1x, Ad1)
    num1 = _edge1(hh1, adv1[:, 0], src2d, dst2d)
    cur1, hh2, xlin2, adv2 = _tcb(num1, xlin1, b1.reshape(1, H1),
                                  delta1.reshape(1, 1), W2, W_lin2,
                                  b_lin2.reshape(1, H2), A2x, Ad2)
    num2 = _edge2(hh2, adv2[:, 0], src2d, dst2d)
    cur2, pq = _tcc(num2, xlin2, b2.reshape(1, H2),
                    delta2.reshape(1, 1), W_post, b_post.reshape(1, 2))
    scores2d = _score(pq[:, 0], pq[:, 1], eli0, eli1)
    scores = scores2d.reshape(-1)[:L]
    return (scores, cur1, cur2)


# speedup vs baseline: 1.1870x; 1.0448x over previous
import functools

import jax
import jax.numpy as jnp
from jax import lax
from jax.experimental import pallas as pl
from jax.experimental.pallas import tpu as pltpu
from jax.experimental.pallas import tpu_sc as plsc

N = 10000
E = 320000
L = 50000
D_IN = 128
H1 = 64
H2 = 32

NC = 2
NS = 16
NW = NC * NS
NP = 10240
RPT = NP // NS

WIN = 128
CH = 16
NWIN = 80
EPW = NWIN * WIN
EPAD = EPW * NW

LPW = 1600
LPAD = LPW * NW

RB = 2048
GRID = -(-N // RB)


def _tca_body(x_ref, w1_ref, wl_ref, bl_ref, ax_ref, ad_ref,
              hh_ref, xl_ref, adv_ref):
    xx = x_ref[...]
    h = jnp.dot(xx, w1_ref[...], preferred_element_type=jnp.float32)
    asd = jnp.dot(h, ax_ref[...], preferred_element_type=jnp.float32)
    hh_ref[...] = jnp.concatenate([h, asd], axis=1)
    xl_ref[...] = jnp.dot(xx, wl_ref[...], preferred_element_type=jnp.float32) + bl_ref[...]
    adv_ref[...] = jnp.dot(h, ad_ref[...], preferred_element_type=jnp.float32)


def _tca(x, W1, Wl1, bl1, A1x, Ad1):
    return pl.pallas_call(
        _tca_body,
        grid=(GRID,),
        in_specs=[
            pl.BlockSpec((RB, D_IN), lambda i: (i, 0)),
            pl.BlockSpec((D_IN, H1), lambda i: (0, 0)),
            pl.BlockSpec((D_IN, H1), lambda i: (0, 0)),
            pl.BlockSpec((1, H1), lambda i: (0, 0)),
            pl.BlockSpec((H1, 16), lambda i: (0, 0)),
            pl.BlockSpec((H1, 8), lambda i: (0, 0)),
        ],
        out_specs=[
            pl.BlockSpec((RB, H1 + 16), lambda i: (i, 0)),
            pl.BlockSpec((RB, H1), lambda i: (i, 0)),
            pl.BlockSpec((RB, 8), lambda i: (i, 0)),
        ],
        out_shape=[
            jax.ShapeDtypeStruct((N, H1 + 16), jnp.float32),
            jax.ShapeDtypeStruct((N, H1), jnp.float32),
            jax.ShapeDtypeStruct((N, 8), jnp.float32),
        ],
    )(x, W1, Wl1, bl1, A1x, Ad1)


def _make_edge_kernel(H):
    HR = H + 16
    QH = H // 16
    RT = N // NS
    mesh = plsc.VectorSubcoreMesh(core_axis_name="c", subcore_axis_name="s")

    @functools.partial(
        pl.kernel,
        mesh=mesh,
        out_type=jax.ShapeDtypeStruct((NC, N, HR), jnp.float32),
        scratch_types=[
            pltpu.VMEM((2, CH, WIN), jnp.int32),
            pltpu.VMEM((2, CH, WIN), jnp.int32),
            pltpu.VMEM((WIN,), jnp.float32),
            pltpu.VMEM((2, WIN, HR), jnp.float32),
            pltpu.VMEM((2, WIN), jnp.float32),
            pltpu.VMEM_SHARED((N, HR), jnp.float32),
            pltpu.VMEM_SHARED((N,), jnp.float32),
            pltpu.VMEM_SHARED((N, HR), jnp.float32),
            pltpu.SemaphoreType.DMA,
            pltpu.SemaphoreType.DMA,
            pltpu.SemaphoreType.DMA,
            pltpu.SemaphoreType.DMA,
        ],
        compiler_params=pltpu.CompilerParams(
            needs_layout_passes=False, use_tc_tiling_on_sc=False),
    )
    def edge_kernel(hh_hbm, ad_hbm, src_hbm, dst_hbm, num_hbm,
                    src_v, dst_v, ex_v, rows_v, adw_v,
                    h_s, ad_s, num_s, gsem0, gsem1, ssem0, ssem1):
        gsem = (gsem0, gsem1)
        ssem = (ssem0, ssem1)
        core = lax.axis_index("c")
        sub = lax.axis_index("s")
        wid = sub * NC + core
        base_win = wid * NWIN

        pltpu.sync_copy(hh_hbm.at[pl.ds(sub * RT, RT)],
                        h_s.at[pl.ds(sub * RT, RT)])

        @pl.when(sub == 0)
        def _stage_ad():
            pltpu.sync_copy(ad_hbm, ad_s)

        zv = jnp.zeros((16,), jnp.float32)

        def zrow_body(r, _):
            for q in range(HR // 16):
                rows_v[0, r, pl.ds(q * 16, 16)] = zv
            return 0

        lax.fori_loop(0, WIN, zrow_body, 0)
        for k in range(RT // 125):
            pltpu.sync_copy(rows_v.at[0, pl.ds(0, 125)],
                            num_s.at[pl.ds(sub * RT + k * 125, 125)])
        plsc.subcore_barrier()

        lane0 = lax.iota(jnp.int32, 16) == 0
        ebase = wid * EPW

        def _issue_gather(w, s):
            cs = lax.rem(lax.div(w, CH), 2)
            wm = lax.rem(w, CH)
            pltpu.async_copy(h_s.at[src_v.at[cs, wm]], rows_v.at[s], gsem[s])
            pltpu.async_copy(ad_s.at[dst_v.at[cs, wm]], adw_v.at[s], gsem[s])

        def _wait_gather(s):
            pltpu.make_async_copy(
                hh_hbm.at[pl.ds(0, WIN)], rows_v.at[s], gsem[s]).wait()
            pltpu.make_async_copy(
                ad_hbm.at[pl.ds(0, WIN)], adw_v.at[s], gsem[s]).wait()

        def _drain_scatter(s):
            pltpu.make_async_copy(
                num_hbm.at[0, pl.ds(0, WIN)], rows_v.at[s], ssem[s]).wait()

        def _load_chunk(c):
            cs = lax.rem(c, 2)
            pltpu.sync_copy(
                src_hbm.at[pl.ds(base_win + c * CH, CH)], src_v.at[cs])
            pltpu.sync_copy(
                dst_hbm.at[pl.ds(base_win + c * CH, CH)], dst_v.at[cs])

        _load_chunk(0)
        _issue_gather(0, 0)

        def pair_body(t, _):
            for s in range(2):
                w = 2 * t + s
                _wait_gather(s)
                cs_w = lax.rem(lax.div(w, CH), 2)
                wm_w = lax.rem(w, CH)

                @pl.when(w + 1 < NWIN)
                def _prefetch():
                    @pl.when(lax.rem(w + 1, CH) == 0)
                    def _chunk():
                        _load_chunk(lax.div(w + 1, CH))

                    @pl.when(w >= 1)
                    def _drain():
                        _drain_scatter(1 - s)

                    _issue_gather(w + 1, 1 - s)

                def grp_body(g, _):
                    off = pl.multiple_of(g * 16, 16)
                    rid = off + lax.iota(jnp.int32, 16)
                    a_s = plsc.load_gather(
                        rows_v, [jnp.full((16,), s, jnp.int32), rid,
                                 jnp.full((16,), H, jnp.int32)])
                    a_d = adw_v[s, pl.ds(off, 16)]
                    e = a_s + a_d
                    e = jnp.where(e >= 0.0, e, e * jnp.float32(0.2))
                    ex = jnp.exp(e)
                    gid = ebase + w * WIN + rid
                    ex = jnp.where(gid < E, ex, jnp.float32(0.0))
                    ex_v[pl.ds(off, 16)] = ex
                    for j in range(16):
                        row = g * 16 + j
                        eb = plsc.load_gather(
                            ex_v, [jnp.full((16,), row, jnp.int32)])
                        for q in range(QH):
                            rows_v[s, row, pl.ds(q * 16, 16)] = (
                                rows_v[s, row, pl.ds(q * 16, 16)] * eb)
                        rows_v[s, row, pl.ds(H, 16)] = jnp.where(
                            lane0, eb, 0.0)
                    return 0

                lax.fori_loop(0, WIN // 16, grp_body, 0)
                pltpu.async_copy(rows_v.at[s], num_s.at[dst_v.at[cs_w, wm_w]],
                                 ssem[s], add=True)
            return 0

        lax.fori_loop(0, NWIN // 2, pair_body, 0)
        _drain_scatter(0)
        _drain_scatter(1)
        plsc.subcore_barrier()
        pltpu.sync_copy(num_s.at[pl.ds(sub * RT, RT)],
                        num_hbm.at[core, pl.ds(sub * RT, RT)])

    return edge_kernel


_edge1 = _make_edge_kernel(H1)
_edge2 = _make_edge_kernel(H2)


def _tcb_body(np_ref, xl_ref, b1_ref, d_ref, w2_ref, wl2_ref,
              bl2_ref, a2x_ref, ad2_ref, cur_ref, hh2_ref, xl2_ref, adv2_ref):
    num = np_ref[0] + np_ref[1]
    h = num[:, 0:H1] / (num[:, H1:H1 + 1] + 1e-16) + b1_ref[...]
    d = d_ref[...]
    cur1 = d * h + (1.0 - d) * xl_ref[...]
    cur_ref[...] = cur1
    h2 = jnp.dot(cur1, w2_ref[...], preferred_element_type=jnp.float32)
    asd2 = jnp.dot(h2, a2x_ref[...], preferred_element_type=jnp.float32)
    hh2_ref[...] = jnp.concatenate([h2, asd2], axis=1)
    xl2_ref[...] = jnp.dot(cur1, wl2_ref[...], preferred_element_type=jnp.float32) + bl2_ref[...]
    adv2_ref[...] = jnp.dot(h2, ad2_ref[...], preferred_element_type=jnp.float32)


def _tcb(num1, xlin1, b1, delta1, W2, Wl2, bl2, A2x, Ad2):
    return pl.pallas_call(
        _tcb_body,
        grid=(GRID,),
        in_specs=[
            pl.BlockSpec((NC, RB, H1 + 16), lambda i: (0, i, 0)),
            pl.BlockSpec((RB, H1), lambda i: (i, 0)),
            pl.BlockSpec((1, H1), lambda i: (0, 0)),
            pl.BlockSpec((1, 1), lambda i: (0, 0)),
            pl.BlockSpec((H1, H2), lambda i: (0, 0)),
            pl.BlockSpec((H1, H2), lambda i: (0, 0)),
            pl.BlockSpec((1, H2), lambda i: (0, 0)),
            pl.BlockSpec((H2, 16), lambda i: (0, 0)),
            pl.BlockSpec((H2, 8), lambda i: (0, 0)),
        ],
        out_specs=[
            pl.BlockSpec((RB, H1), lambda i: (i, 0)),
            pl.BlockSpec((RB, H2 + 16), lambda i: (i, 0)),
            pl.BlockSpec((RB, H2), lambda i: (i, 0)),
            pl.BlockSpec((RB, 8), lambda i: (i, 0)),
        ],
        out_shape=[
            jax.ShapeDtypeStruct((N, H1), jnp.float32),
            jax.ShapeDtypeStruct((N, H2 + 16), jnp.float32),
            jax.ShapeDtypeStruct((N, H2), jnp.float32),
            jax.ShapeDtypeStruct((N, 8), jnp.float32),
        ],
    )(num1, xlin1, b1, delta1, W2, Wl2, bl2, A2x, Ad2)


def _tcc_body(np_ref, xl_ref, b2_ref, d_ref, wp_ref, bp_ref,
              cur_ref, pq_ref):
    num = np_ref[0] + np_ref[1]
    h = num[:, 0:H2] / (num[:, H2:H2 + 1] + 1e-16) + b2_ref[...]
    d = d_ref[...]
    cur2 = d * h + (1.0 - d) * xl_ref[...]
    cur_ref[...] = cur2
    t1 = jnp.dot(cur2, wp_ref[0:H2, :], preferred_element_type=jnp.float32)
    t2 = jnp.dot(cur2, wp_ref[H2:2 * H2, :], preferred_element_type=jnp.float32)
    bp = bp_ref[...]
    c = bp[:, 0:1] + bp[:, 1:2]
    p = t1[:, 0:1] + t1[:, 1:2] + c
    q = t2[:, 0:1] + t2[:, 1:2]
    pq_ref[...] = jnp.concatenate(
        [p, q, jnp.zeros((p.shape[0], 6), jnp.float32)], axis=1)


def _tcc(num2, xlin2, b2, delta2, Wp, bp):
    return pl.pallas_call(
        _tcc_body,
        grid=(GRID,),
        in_specs=[
            pl.BlockSpec((NC, RB, H2 + 16), lambda i: (0, i, 0)),
            pl.BlockSpec((RB, H2), lambda i: (i, 0)),
            pl.BlockSpec((1, H2), lambda i: (0, 0)),
            pl.BlockSpec((1, 1), lambda i: (0, 0)),
            pl.BlockSpec((2 * H2, 2), lambda i: (0, 0)),
            pl.BlockSpec((1, 2), lambda i: (0, 0)),
        ],
        out_specs=[
            pl.BlockSpec((RB, H2), lambda i: (i, 0)),
            pl.BlockSpec((RB, 8), lambda i: (i, 0)),
        ],
        out_shape=[
            jax.ShapeDtypeStruct((N, H2), jnp.float32),
            jax.ShapeDtypeStruct((N, 8), jnp.float32),
        ],
    )(num2, xlin2, b2, delta2, Wp, bp)


def _make_score_kernel():
    mesh = plsc.VectorSubcoreMesh(core_axis_name="c", subcore_axis_name="s")

    @functools.partial(
        pl.kernel,
        mesh=mesh,
        out_type=jax.ShapeDtypeStruct((NW, 1, LPW), jnp.float32),
        scratch_types=[
            pltpu.VMEM((N,), jnp.float32),
            pltpu.VMEM((N,), jnp.float32),
            pltpu.VMEM((1, LPW), jnp.int32),
            pltpu.VMEM((1, LPW), jnp.int32),
            pltpu.VMEM((1, LPW), jnp.float32),
        ],
        compiler_params=pltpu.CompilerParams(
            needs_layout_passes=False, use_tc_tiling_on_sc=False),
    )
    def score_kernel(p_hbm, q_hbm, i0_hbm, i1_hbm, out_hbm,
                     p_v, q_v, i0_v, i1_v, out_v):
        core = lax.axis_index("c")
        sub = lax.axis_index("s")
        wid = sub * NC + core
        pltpu.sync_copy(p_hbm, p_v)
        pltpu.sync_copy(q_hbm, q_v)
        pltpu.sync_copy(i0_hbm.at[wid], i0_v)
        pltpu.sync_copy(i1_hbm.at[wid], i1_v)

        def body(g, _):
            off = pl.multiple_of(g * 16, 16)
            idx0 = i0_v[0, pl.ds(off, 16)]
            idx1 = i1_v[0, pl.ds(off, 16)]
            out_v[0, pl.ds(off, 16)] = (
                plsc.load_gather(p_v, [idx0]) + plsc.load_gather(q_v, [idx1]))
            return 0

        lax.fori_loop(0, LPW // 16, body, 0)
        pltpu.sync_copy(out_v, out_hbm.at[wid])

    return score_kernel


_score = _make_score_kernel()


def kernel(x, edge_index, edge_label_index, snap, W_lin1, b_lin1, W1, a_src1,
           a_dst1, b1, Wsem1, bsem1, qsem1, delta1, W_lin2, b_lin2, W2,
           a_src2, a_dst2, b2, Wsem2, bsem2, qsem2, delta2, W_post, b_post):
    f32 = jnp.float32
    pad_e = (jnp.arange(EPAD - E, dtype=jnp.int32) * 37) % N
    src2d = jnp.concatenate([edge_index[0], pad_e]).reshape(NW * NWIN, WIN)
    dst2d = jnp.concatenate([edge_index[1], pad_e]).reshape(NW * NWIN, WIN)
    pad_l = (jnp.arange(LPAD - L, dtype=jnp.int32) * 53) % N
    eli0 = jnp.concatenate([edge_label_index[0], pad_l]).reshape(NW, 1, LPW)
    eli1 = jnp.concatenate([edge_label_index[1], pad_l]).reshape(NW, 1, LPW)

    A1x = jnp.zeros((H1, 16), f32).at[:, 0].set(a_src1)
    Ad1 = jnp.zeros((H1, 8), f32).at[:, 0].set(a_dst1)
    A2x = jnp.zeros((H2, 16), f32).at[:, 0].set(a_src2)
    Ad2 = jnp.zeros((H2, 8), f32).at[:, 0].set(a_dst2)

    hh1, xlin1, adv1 = _tca(x, W1, W_lin1, b_lin1.reshape(1, H1), A1x, Ad1)
    num1 = _edge1(hh1, adv1[:, 0], src2d, dst2d)
    cur1, hh2, xlin2, adv2 = _tcb(num1, xlin1, b1.reshape(1, H1),
                                  delta1.reshape(1, 1), W2, W_lin2,
                                  b_lin2.reshape(1, H2), A2x, Ad2)
    num2 = _edge2(hh2, adv2[:, 0], src2d, dst2d)
    cur2, pq = _tcc(num2, xlin2, b2.reshape(1, H2),
                    delta2.reshape(1, 1), W_post, b_post.reshape(1, 2))
    scores2d = _score(pq[:, 0], pq[:, 1], eli0, eli1)
    scores = scores2d.reshape(-1)[:L]
    return (scores, cur1, cur2)
